# initial kernel scaffold (unmeasured)
import jax
import jax.numpy as jnp
from jax import lax
from jax.experimental import pallas as pl
from jax.experimental.pallas import tpu as pltpu


def kernel(
    x,
):
    def body(*refs):
        pass

    out_shape = jax.ShapeDtypeStruct(..., jnp.float32)
    return pl.pallas_call(body, out_shape=out_shape)(...)


# baseline (device time: 214626 ns/iter reference)
import jax
import jax.numpy as jnp
from jax import lax
from jax.experimental import pallas as pl
from jax.experimental.pallas import tpu as pltpu


def kernel(x):
    m, n = x.shape
    xb = x.astype(jnp.bfloat16)

    def body(x_ref, out_ref, comm_ref, send_sem, recv_sem):
        my_x = lax.axis_index("x")
        my_y = lax.axis_index("y")
        my_z = lax.axis_index("z")
        partner = (my_x, my_y, 1 - my_z)

        barrier_sem = pltpu.get_barrier_semaphore()
        pl.semaphore_signal(
            barrier_sem,
            inc=1,
            device_id=partner,
            device_id_type=pl.DeviceIdType.MESH,
        )
        pl.semaphore_wait(barrier_sem, 1)

        rdma = pltpu.make_async_remote_copy(
            src_ref=x_ref,
            dst_ref=comm_ref,
            send_sem=send_sem,
            recv_sem=recv_sem,
            device_id=partner,
            device_id_type=pl.DeviceIdType.MESH,
        )
        rdma.start()
        rdma.wait()

        out_ref[...] = x_ref[...] + comm_ref[...]

    return pl.pallas_call(
        body,
        out_shape=jax.ShapeDtypeStruct((m, n), jnp.bfloat16),
        in_specs=[pl.BlockSpec(memory_space=pltpu.VMEM)],
        out_specs=pl.BlockSpec(memory_space=pltpu.VMEM),
        scratch_shapes=[
            pltpu.VMEM((m, n), jnp.bfloat16),
            pltpu.SemaphoreType.DMA,
            pltpu.SemaphoreType.DMA,
        ],
        compiler_params=pltpu.CompilerParams(collective_id=0),
    )(xb)


# device time: 173019 ns/iter; 1.2405x vs baseline; 1.2405x over previous
import jax
import jax.numpy as jnp
from jax import lax
from jax.experimental import pallas as pl
from jax.experimental.pallas import tpu as pltpu


def kernel(x):
    m, n = x.shape
    qm = m // 4
    xb = x.astype(jnp.bfloat16)

    def body(x_ref, out_ref, comm_ref, send_sems, recv_sems):
        my_x = lax.axis_index("x")
        my_y = lax.axis_index("y")
        my_z = lax.axis_index("z")
        z_nbr = (my_x, my_y, 1 - my_z)
        x_nbr = (1 - my_x, my_y, my_z)
        y_nbr = (my_x, 1 - my_y, my_z)

        c = 2 * my_x + my_y
        cx = 2 * (1 - my_x) + my_y

        barrier_sem = pltpu.get_barrier_semaphore()
        for nbr in (z_nbr, x_nbr, y_nbr):
            pl.semaphore_signal(
                barrier_sem,
                inc=1,
                device_id=nbr,
                device_id_type=pl.DeviceIdType.MESH,
            )
        pl.semaphore_wait(barrier_sem, 3)

        p1 = pltpu.make_async_remote_copy(
            src_ref=x_ref.at[pl.ds(c * qm, qm)],
            dst_ref=comm_ref,
            send_sem=send_sems.at[0],
            recv_sem=recv_sems.at[0],
            device_id=z_nbr,
            device_id_type=pl.DeviceIdType.MESH,
        )
        p1.start()
        p1.wait()
        out_ref[pl.ds(c * qm, qm), :] = (
            x_ref[pl.ds(c * qm, qm), :] + comm_ref[...]
        )

        p2a = pltpu.make_async_remote_copy(
            src_ref=out_ref.at[pl.ds(c * qm, qm)],
            dst_ref=out_ref.at[pl.ds(c * qm, qm)],
            send_sem=send_sems.at[1],
            recv_sem=recv_sems.at[1],
            device_id=x_nbr,
            device_id_type=pl.DeviceIdType.MESH,
        )
        p2a.start()

        p2b1 = pltpu.make_async_remote_copy(
            src_ref=out_ref.at[pl.ds(c * qm, qm)],
            dst_ref=out_ref.at[pl.ds(c * qm, qm)],
            send_sem=send_sems.at[2],
            recv_sem=recv_sems.at[2],
            device_id=y_nbr,
            device_id_type=pl.DeviceIdType.MESH,
        )
        p2b1.start()

        p2a.wait()
        p2b2 = pltpu.make_async_remote_copy(
            src_ref=out_ref.at[pl.ds(cx * qm, qm)],
            dst_ref=out_ref.at[pl.ds(cx * qm, qm)],
            send_sem=send_sems.at[3],
            recv_sem=recv_sems.at[3],
            device_id=y_nbr,
            device_id_type=pl.DeviceIdType.MESH,
        )
        p2b2.start()
        p2b1.wait()
        p2b2.wait()

    return pl.pallas_call(
        body,
        out_shape=jax.ShapeDtypeStruct((m, n), jnp.bfloat16),
        in_specs=[pl.BlockSpec(memory_space=pltpu.VMEM)],
        out_specs=pl.BlockSpec(memory_space=pltpu.VMEM),
        scratch_shapes=[
            pltpu.VMEM((qm, n), jnp.bfloat16),
            pltpu.SemaphoreType.DMA((4,)),
            pltpu.SemaphoreType.DMA((4,)),
        ],
        compiler_params=pltpu.CompilerParams(collective_id=0),
    )(xb)


# device time: 109485 ns/iter; 1.9603x vs baseline; 1.5803x over previous
import jax
import jax.numpy as jnp
from jax import lax
from jax.experimental import pallas as pl
from jax.experimental.pallas import tpu as pltpu

CH = 256
NCH = 32
PIECES = {0: (0, 11), 1: (11, 22), 2: (22, 32)}
PIECE_OF = {(0, 0): 0, (0, 1): 1, (1, 0): 2, (1, 1): 0}
MAXCH = 11


def kernel(x):
    m, n = x.shape
    xb = x.astype(jnp.bfloat16)

    def body(x_ref, out_ref, comm_ref, sz, rz, sx, rx, sy, ry):
        my_x = lax.axis_index("x")
        my_y = lax.axis_index("y")
        my_z = lax.axis_index("z")

        barrier_sem = pltpu.get_barrier_semaphore()
        for nbr in (
            (my_x, my_y, 1 - my_z),
            (1 - my_x, my_y, my_z),
            (my_x, 1 - my_y, my_z),
        ):
            pl.semaphore_signal(
                barrier_sem,
                inc=1,
                device_id=nbr,
                device_id_type=pl.DeviceIdType.MESH,
            )
        pl.semaphore_wait(barrier_sem, 3)

        def chunk_rows(g):
            return pl.ds(g * CH, CH)

        def emit_column(cx, cy):
            own_p = PIECE_OF[(cx, cy)]
            xin_p = PIECE_OF[(1 - cx, cy)]
            y_nbr_c = (cx, 1 - cy)
            diag = cx == cy
            ynbr_diag = cx == (1 - cy)
            yin_p = (
                PIECE_OF[(1 - cx, 1 - cy)]
                if ynbr_diag
                else PIECE_OF[y_nbr_c]
            )
            own_s, own_e = PIECES[own_p]
            xin_s, xin_e = PIECES[xin_p]
            yin_s, yin_e = PIECES[yin_p]
            n_own = own_e - own_s
            n_xin = xin_e - xin_s
            n_yin = yin_e - yin_s

            z_dev = (cx, cy, 1 - my_z)
            x_dev = (1 - cx, cy, my_z)
            y_dev = (cx, 1 - cy, my_z)

            z_rdmas = []
            for j in range(n_own):
                g = own_s + j
                r = pltpu.make_async_remote_copy(
                    src_ref=x_ref.at[chunk_rows(g)],
                    dst_ref=comm_ref.at[pl.ds(j * CH, CH)],
                    send_sem=sz.at[j],
                    recv_sem=rz.at[j],
                    device_id=z_dev,
                    device_id_type=pl.DeviceIdType.MESH,
                )
                r.start()
                z_rdmas.append(r)

            x_recvs = [
                pltpu.make_async_remote_copy(
                    src_ref=out_ref.at[chunk_rows(xin_s + j)],
                    dst_ref=out_ref.at[chunk_rows(xin_s + j)],
                    send_sem=sx.at[j],
                    recv_sem=rx.at[j],
                    device_id=x_dev,
                    device_id_type=pl.DeviceIdType.MESH,
                )
                for j in range(n_xin)
            ]
            y_recvs = [
                pltpu.make_async_remote_copy(
                    src_ref=out_ref.at[chunk_rows(yin_s + j)],
                    dst_ref=out_ref.at[chunk_rows(yin_s + j)],
                    send_sem=sy.at[j],
                    recv_sem=ry.at[j],
                    device_id=y_dev,
                    device_id_type=pl.DeviceIdType.MESH,
                )
                for j in range(n_yin)
            ]

            sends = []
            for t in range(max(n_own, n_xin if diag else 0)):
                if t < n_own:
                    g = own_s + t
                    z_rdmas[t].wait()
                    out_ref[chunk_rows(g), :] = (
                        x_ref[chunk_rows(g), :]
                        + comm_ref[pl.ds(t * CH, CH), :]
                    )
                    xs = pltpu.make_async_remote_copy(
                        src_ref=out_ref.at[chunk_rows(g)],
                        dst_ref=out_ref.at[chunk_rows(g)],
                        send_sem=sx.at[t],
                        recv_sem=rx.at[t],
                        device_id=x_dev,
                        device_id_type=pl.DeviceIdType.MESH,
                    )
                    xs.start()
                    sends.append(xs)
                    if not diag:
                        ys = pltpu.make_async_remote_copy(
                            src_ref=out_ref.at[chunk_rows(g)],
                            dst_ref=out_ref.at[chunk_rows(g)],
                            send_sem=sy.at[t],
                            recv_sem=ry.at[t],
                            device_id=y_dev,
                            device_id_type=pl.DeviceIdType.MESH,
                        )
                        ys.start()
                        sends.append(ys)
                if diag and t < n_xin:
                    g = xin_s + t
                    x_recvs[t].wait_recv()
                    ys = pltpu.make_async_remote_copy(
                        src_ref=out_ref.at[chunk_rows(g)],
                        dst_ref=out_ref.at[chunk_rows(g)],
                        send_sem=sy.at[t],
                        recv_sem=ry.at[t],
                        device_id=y_dev,
                        device_id_type=pl.DeviceIdType.MESH,
                    )
                    ys.start()
                    sends.append(ys)

            if not diag:
                for r in x_recvs:
                    r.wait_recv()
            for r in y_recvs:
                r.wait_recv()
            for s in sends:
                s.wait_send()

        for cx in (0, 1):
            for cy in (0, 1):

                @pl.when((my_x == cx) & (my_y == cy))
                def _(cx=cx, cy=cy):
                    emit_column(cx, cy)

    return pl.pallas_call(
        body,
        out_shape=jax.ShapeDtypeStruct((m, n), jnp.bfloat16),
        in_specs=[pl.BlockSpec(memory_space=pltpu.VMEM)],
        out_specs=pl.BlockSpec(memory_space=pltpu.VMEM),
        scratch_shapes=[
            pltpu.VMEM((MAXCH * CH, n), jnp.bfloat16),
            pltpu.SemaphoreType.DMA((MAXCH,)),
            pltpu.SemaphoreType.DMA((MAXCH,)),
            pltpu.SemaphoreType.DMA((MAXCH,)),
            pltpu.SemaphoreType.DMA((MAXCH,)),
            pltpu.SemaphoreType.DMA((MAXCH,)),
            pltpu.SemaphoreType.DMA((MAXCH,)),
        ],
        compiler_params=pltpu.CompilerParams(collective_id=0),
    )(xb)
